# Initial kernel scaffold; baseline (speedup 1.0000x reference)
import jax
import jax.numpy as jnp
from jax.experimental import pallas as pl


def kernel(feature, A, W1, b1, W2, b2, weight_b):
    raise NotImplementedError("write your pallas kernel here")


# speedup vs baseline: 3.3653x; 3.3653x over previous
import functools

import jax
import jax.numpy as jnp
from jax.experimental import pallas as pl

N = 4096
BM = 256


def _scaled_rhs_kernel(x_ref, w_ref, wb_ref, out_ref):
    z = jnp.dot(x_ref[...], w_ref[...], preferred_element_type=jnp.float32)
    out_ref[0] = wb_ref[0, 0] * z
    out_ref[1] = wb_ref[1, 0] * z


def _pass1_kernel(a_ref, zs_ref, b1_ref, u1_ref):
    acc = jnp.dot(a_ref[0], zs_ref[0], preferred_element_type=jnp.float32)
    acc += jnp.dot(a_ref[1], zs_ref[1], preferred_element_type=jnp.float32)
    u1_ref[...] = jnp.maximum(acc + b1_ref[...], 0.0)


def _pass2_kernel(a_ref, vs_ref, u1_ref, b2_ref, out_ref):
    acc = jnp.dot(a_ref[0], vs_ref[0], preferred_element_type=jnp.float32)
    acc += jnp.dot(a_ref[1], vs_ref[1], preferred_element_type=jnp.float32)
    out_ref[...] = 0.5 * (u1_ref[...] + acc + b2_ref[...])


@jax.jit
def kernel(feature, A, W1, b1, W2, b2, weight_b):
    n = A.shape[1]
    hid = W1.shape[1]
    out_dim = W2.shape[1]
    grid = (n // BM,)

    a_spec = pl.BlockSpec((2, BM, n), lambda i: (0, i, 0))
    full2 = lambda d: pl.BlockSpec((2, n, d), lambda i: (0, 0, 0))
    row_spec = lambda d: pl.BlockSpec((BM, d), lambda i: (i, 0))
    bias_spec = lambda d: pl.BlockSpec((1, d), lambda i: (0, 0))

    zs = pl.pallas_call(
        _scaled_rhs_kernel,
        out_shape=jax.ShapeDtypeStruct((2, n, hid), jnp.float32),
    )(feature, W1, weight_b)

    u1 = pl.pallas_call(
        _pass1_kernel,
        grid=grid,
        in_specs=[a_spec, full2(hid), bias_spec(hid)],
        out_specs=row_spec(hid),
        out_shape=jax.ShapeDtypeStruct((n, hid), jnp.float32),
    )(A, zs, b1.reshape(1, hid))

    vs = pl.pallas_call(
        _scaled_rhs_kernel,
        out_shape=jax.ShapeDtypeStruct((2, n, out_dim), jnp.float32),
    )(u1, W2, weight_b)

    out = pl.pallas_call(
        _pass2_kernel,
        grid=grid,
        in_specs=[a_spec, full2(out_dim), row_spec(hid), bias_spec(out_dim)],
        out_specs=row_spec(out_dim),
        out_shape=jax.ShapeDtypeStruct((n, out_dim), jnp.float32),
    )(A, vs, u1, b2.reshape(1, out_dim))

    return out
